# Initial kernel scaffold; baseline (speedup 1.0000x reference)
import jax
import jax.numpy as jnp
from jax.experimental import pallas as pl


def kernel(x, edge_index, batch, W1a, b1a, g1, bt1, W1b, b1b, W2a, b2a, g2, bt2, W2b, b2b, W3a, b3a, g3, bt3, W3b, b3b, Wl1, bl1, Wl2, bl2):
    raise NotImplementedError("write your pallas kernel here")


# speedup vs baseline: 4.3961x; 4.3961x over previous
import functools

import jax
import jax.numpy as jnp
from jax import lax
from jax.experimental import pallas as pl
from jax.experimental.pallas import tpu as pltpu
from jax.experimental.pallas import tpu_sc as plsc

_N = 50000
_E = 800000
_G = 64
_D = 64
_BN_EPS = 1e-5

_NS = 16
_H = 25088
_NPAD = 2 * _H
_DUMMY = _H
_HACC = _H + 8
_EPT = 50176
_EPAD = _EPT * _NS
_EROWS = _EPAD // 128
_RPT = _H // _NS
_TROWS = _EPT // 128

_BLK = 512
_NBLK = _NPAD // _BLK


def _seg_half_sum(W, CH):
    mesh = plsc.VectorSubcoreMesh(core_axis_name="c", subcore_axis_name="s")
    SUB = CH // 128
    CHUNKS = _EPT // CH

    @functools.partial(
        pl.kernel,
        mesh=mesh,
        compiler_params=pltpu.CompilerParams(use_tc_tiling_on_sc=False),
        out_type=jax.ShapeDtypeStruct((_NPAD, W), jnp.float32),
        scratch_types=[
            pltpu.VMEM_SHARED((_HACC, W), jnp.float32),
            pltpu.VMEM((SUB, 128), jnp.int32),
            pltpu.VMEM((SUB, 128), jnp.int32),
            pltpu.VMEM((128,), jnp.int32),
            pltpu.VMEM((CH, W), jnp.float32),
            pltpu.SemaphoreType.DMA,
        ],
    )
    def seg(h_hbm, src_hbm, dst_hbm, out_hbm, acc, src_v, dst_v, ldst_v,
            rows_v, sem):
        core = lax.axis_index("c")
        sub = lax.axis_index("s")
        base = core * _H
        pltpu.sync_copy(h_hbm.at[pl.ds(base + sub * _RPT, _RPT)],
                        acc.at[pl.ds(sub * _RPT, _RPT)])
        plsc.subcore_barrier()
        row0 = sub * _TROWS

        @pl.loop(0, CHUNKS)
        def _chunk(c):
            r = row0 + c * SUB
            pltpu.sync_copy(src_hbm.at[pl.ds(r, SUB)], src_v)
            pltpu.sync_copy(dst_hbm.at[pl.ds(r, SUB)], dst_v)
            copies = [
                pltpu.async_copy(h_hbm.at[src_v.at[j]],
                                 rows_v.at[pl.ds(j * 128, 128)], sem)
                for j in range(SUB)
            ]
            for cp in copies:
                cp.wait()
            for j in range(SUB):
                drow = dst_v.at[j]
                for i in range(8):
                    d = drow[pl.ds(i * 16, 16)]
                    ok = (d >= base) & (d < base + _H)
                    ldst_v[pl.ds(i * 16, 16)] = jnp.where(ok, d - base, _DUMMY)
                pltpu.sync_copy(rows_v.at[pl.ds(j * 128, 128)],
                                acc.at[ldst_v], add=True)

        plsc.subcore_barrier()
        pltpu.sync_copy(acc.at[pl.ds(sub * _RPT, _RPT)],
                        out_hbm.at[pl.ds(base + sub * _RPT, _RPT)])

    return seg


_seg16 = _seg_half_sum(16, 1024)
_seg64 = _seg_half_sum(64, 256)


def _mlp(h, Wa, ba, Wb, bb, K):
    def body(h_ref, wa_ref, ba_ref, wb_ref, bb_ref, o_ref):
        t = jnp.dot(h_ref[...], wa_ref[...], preferred_element_type=jnp.float32)
        t = jnp.maximum(t + ba_ref[...], 0.0)
        t = jnp.dot(t, wb_ref[...], preferred_element_type=jnp.float32)
        o_ref[...] = jnp.maximum(t + bb_ref[...], 0.0)

    return pl.pallas_call(
        body,
        grid=(_NBLK,),
        in_specs=[
            pl.BlockSpec((_BLK, K), lambda i: (i, 0)),
            pl.BlockSpec((K, _D), lambda i: (0, 0)),
            pl.BlockSpec((1, _D), lambda i: (0, 0)),
            pl.BlockSpec((_D, _D), lambda i: (0, 0)),
            pl.BlockSpec((1, _D), lambda i: (0, 0)),
        ],
        out_specs=pl.BlockSpec((_BLK, _D), lambda i: (i, 0)),
        out_shape=jax.ShapeDtypeStruct((_NPAD, _D), jnp.float32),
    )(h, Wa, ba, Wb, bb)


def _mlp_pool_head(s3, batch3d, Wa, ba, Wb, bb, Wl1, bl1, Wl2, bl2):
    def body(b_ref, h_ref, wa_ref, ba_ref, wb_ref, bb_ref, wl1_ref, bl1_ref,
             wl2_ref, bl2_ref, o_ref, acc_ref):
        step = pl.program_id(0)

        @pl.when(step == 0)
        def _():
            acc_ref[...] = jnp.zeros_like(acc_ref)

        t = jnp.dot(h_ref[...], wa_ref[...], preferred_element_type=jnp.float32)
        t = jnp.maximum(t + ba_ref[...], 0.0)
        t = jnp.dot(t, wb_ref[...], preferred_element_type=jnp.float32)
        h3 = jnp.maximum(t + bb_ref[...], 0.0)
        bids = b_ref[0]
        gids = lax.broadcasted_iota(jnp.int32, (_G, _BLK), 0)
        onehot_t = (jnp.broadcast_to(bids, (_G, _BLK)) == gids).astype(
            jnp.float32)
        acc_ref[...] += jnp.dot(onehot_t, h3,
                                preferred_element_type=jnp.float32)

        @pl.when(step == _NBLK - 1)
        def _():
            hg = acc_ref[...]
            u = jnp.dot(hg, wl1_ref[...], preferred_element_type=jnp.float32)
            u = jnp.maximum(u + bl1_ref[...], 0.0)
            o_ref[...] = jnp.dot(
                u, wl2_ref[...], preferred_element_type=jnp.float32) + bl2_ref[...]

    return pl.pallas_call(
        body,
        grid=(_NBLK,),
        in_specs=[
            pl.BlockSpec((1, 1, _BLK), lambda i: (i, 0, 0)),
            pl.BlockSpec((_BLK, _D), lambda i: (i, 0)),
            pl.BlockSpec((_D, _D), lambda i: (0, 0)),
            pl.BlockSpec((1, _D), lambda i: (0, 0)),
            pl.BlockSpec((_D, _D), lambda i: (0, 0)),
            pl.BlockSpec((1, _D), lambda i: (0, 0)),
            pl.BlockSpec((_D, _D), lambda i: (0, 0)),
            pl.BlockSpec((1, _D), lambda i: (0, 0)),
            pl.BlockSpec((_D, 1), lambda i: (0, 0)),
            pl.BlockSpec((1, 1), lambda i: (0, 0)),
        ],
        out_specs=pl.BlockSpec((_G, 1), lambda i: (0, 0)),
        out_shape=jax.ShapeDtypeStruct((_G, 1), jnp.float32),
        scratch_shapes=[pltpu.VMEM((_G, _D), jnp.float32)],
    )(batch3d, s3, Wa, ba, Wb, bb, Wl1, bl1, Wl2, bl2)


def kernel(x, edge_index, batch, W1a, b1a, g1, bt1, W1b, b1b, W2a, b2a, g2,
           bt2, W2b, b2b, W3a, b3a, g3, bt3, W3b, b3b, Wl1, bl1, Wl2, bl2):
    f32 = jnp.float32
    scale = 1.0 / jnp.sqrt(jnp.float32(1.0 + _BN_EPS))

    def fold(Wa, ba, g, bt, K):
        s = g * scale
        Wf = Wa * s[None, :]
        bf = ba * s + bt
        Wp = jnp.zeros((K, _D), f32).at[: Wa.shape[0]].set(Wf)
        return Wp, bf.reshape(1, _D)

    W1a_p, b1a_p = fold(W1a, b1a, g1, bt1, 16)
    W2a_p, b2a_p = fold(W2a, b2a, g2, bt2, _D)
    W3a_p, b3a_p = fold(W3a, b3a, g3, bt3, _D)

    h0 = jnp.zeros((_NPAD, 16), f32).at[:_N, :11].set(x)
    src = edge_index[0]
    dst = edge_index[1]
    src_p = jnp.zeros((_EPAD,), jnp.int32).at[:_E].set(src).reshape(_EROWS, 128)
    dst_p = jnp.full((_EPAD,), -1, jnp.int32).at[:_E].set(dst).reshape(
        _EROWS, 128)
    batch3d = jnp.full((_NPAD,), _G, jnp.int32).at[:_N].set(batch).reshape(
        _NBLK, 1, _BLK)

    s1 = _seg16(h0, src_p, dst_p)
    h1 = _mlp(s1, W1a_p, b1a_p, W1b, b1b.reshape(1, _D), 16)
    s2 = _seg64(h1, src_p, dst_p)
    h2 = _mlp(s2, W2a_p, b2a_p, W2b, b2b.reshape(1, _D), _D)
    s3 = _seg64(h2, src_p, dst_p)
    return _mlp_pool_head(s3, batch3d, W3a_p, b3a_p, W3b,
                          b3b.reshape(1, _D), Wl1, bl1.reshape(1, _D), Wl2,
                          bl2.reshape(1, 1))


# speedup vs baseline: 4.7183x; 1.0733x over previous
import functools

import jax
import jax.numpy as jnp
from jax import lax
from jax.experimental import pallas as pl
from jax.experimental.pallas import tpu as pltpu
from jax.experimental.pallas import tpu_sc as plsc

_N = 50000
_E = 800000
_G = 64
_D = 64
_BN_EPS = 1e-5

_NS = 16
_H = 25088
_NPAD = 2 * _H
_DUMMY = _H
_HACC = _H + 8
_EPT = 50176
_EPAD = _EPT * _NS
_EROWS = _EPAD // 128
_EROWS_PAD = _EROWS + 8
_RPT = _H // _NS
_TROWS = _EPT // 128

_BLK = 512
_NBLK = _NPAD // _BLK


def _seg_half_sum(W, CH):
    mesh = plsc.VectorSubcoreMesh(core_axis_name="c", subcore_axis_name="s")
    SUB = CH // 128
    CHUNKS = _EPT // CH
    assert CHUNKS % 2 == 0

    @functools.partial(
        pl.kernel,
        mesh=mesh,
        compiler_params=pltpu.CompilerParams(use_tc_tiling_on_sc=False),
        out_type=jax.ShapeDtypeStruct((_NPAD, W), jnp.float32),
        scratch_types=[
            pltpu.VMEM_SHARED((_HACC, W), jnp.float32),
            pltpu.VMEM((SUB, 2, 128), jnp.int32),
            pltpu.VMEM((SUB, 2, 128), jnp.int32),
            pltpu.VMEM((CH, W), jnp.float32),
            pltpu.VMEM((CH, W), jnp.float32),
            pltpu.VMEM((128,), jnp.int32),
            pltpu.SemaphoreType.DMA,
            pltpu.SemaphoreType.DMA,
            pltpu.SemaphoreType.DMA,
            pltpu.SemaphoreType.DMA,
        ],
    )
    def seg(h_hbm, edg_hbm, out_hbm, acc, idx0, idx1, rows0, rows1, ldst_v,
            sg0, sg1, si0, si1):
        core = lax.axis_index("c")
        sub = lax.axis_index("s")
        base = core * _H
        pltpu.sync_copy(h_hbm.at[pl.ds(base + sub * _RPT, _RPT)],
                        acc.at[pl.ds(sub * _RPT, _RPT)])
        plsc.subcore_barrier()
        row0 = sub * _TROWS
        idxb, rowsb, sg, si = [idx0, idx1], [rows0, rows1], [sg0, sg1], [si0, si1]

        def idx_rows(c):
            return edg_hbm.at[pl.ds(row0 + c * SUB, SUB)]

        def fire_gathers(b):
            for j in range(SUB):
                pltpu.async_copy(h_hbm.at[idxb[b].at[j].at[0]],
                                 rowsb[b].at[pl.ds(j * 128, 128)], sg[b])

        def wait_gathers(b):
            for j in range(SUB):
                pltpu.make_async_copy(h_hbm.at[idxb[b].at[j].at[0]],
                                      rowsb[b].at[pl.ds(j * 128, 128)],
                                      sg[b]).wait()

        pltpu.sync_copy(idx_rows(0), idx0)
        fire_gathers(0)
        pltpu.async_copy(idx_rows(1), idx1, si1)

        def step(c, cur):
            nxt = 1 - cur
            pltpu.make_async_copy(idx_rows(c + 1), idxb[nxt], si[nxt]).wait()
            fire_gathers(nxt)
            wait_gathers(cur)
            for j in range(SUB):
                drow = idxb[cur].at[j].at[1]
                for i in range(8):
                    d = drow[pl.ds(i * 16, 16)]
                    ok = (d >= base) & (d < base + _H)
                    ldst_v[pl.ds(i * 16, 16)] = jnp.where(ok, d - base, _DUMMY)
                pltpu.sync_copy(rowsb[cur].at[pl.ds(j * 128, 128)],
                                acc.at[ldst_v], add=True)
            pltpu.async_copy(idx_rows(c + 2), idxb[cur], si[cur])

        @pl.loop(0, CHUNKS // 2)
        def _pair(t):
            c = t * 2
            step(c, 0)
            step(c + 1, 1)

        wait_gathers(0)
        pltpu.make_async_copy(idx_rows(CHUNKS + 1), idx1, si1).wait()

        plsc.subcore_barrier()
        pltpu.sync_copy(acc.at[pl.ds(sub * _RPT, _RPT)],
                        out_hbm.at[pl.ds(base + sub * _RPT, _RPT)])

    return seg


_seg16 = _seg_half_sum(16, 512)
_seg64 = _seg_half_sum(64, 128)


def _dot(a, b):
    return jnp.dot(a, b, preferred_element_type=jnp.float32)


def _mlp(h, Wa, ba, g, bt, Wb, bb, K):
    def body(h_ref, wa_ref, ba_ref, g_ref, bt_ref, wb_ref, bb_ref, o_ref):
        t = _dot(h_ref[...], wa_ref[...]) + ba_ref[...]
        t = t / jnp.sqrt(jnp.float32(1.0 + _BN_EPS)) * g_ref[...] + bt_ref[...]
        t = jnp.maximum(t, 0.0)
        t = _dot(t, wb_ref[...]) + bb_ref[...]
        o_ref[...] = jnp.maximum(t, 0.0)

    vec = pl.BlockSpec((1, _D), lambda i: (0, 0))
    return pl.pallas_call(
        body,
        grid=(_NBLK,),
        in_specs=[
            pl.BlockSpec((_BLK, K), lambda i: (i, 0)),
            pl.BlockSpec((K, _D), lambda i: (0, 0)),
            vec, vec, vec,
            pl.BlockSpec((_D, _D), lambda i: (0, 0)),
            vec,
        ],
        out_specs=pl.BlockSpec((_BLK, _D), lambda i: (i, 0)),
        out_shape=jax.ShapeDtypeStruct((_NPAD, _D), jnp.float32),
    )(h, Wa, ba, g, bt, Wb, bb)


def _mlp_pool_head(s3, batch3d, Wa, ba, g, bt, Wb, bb, Wl1, bl1, Wl2, bl2):
    def body(b_ref, h_ref, wa_ref, ba_ref, g_ref, bt_ref, wb_ref, bb_ref,
             wl1_ref, bl1_ref, wl2_ref, bl2_ref, o_ref, acc_ref):
        step = pl.program_id(0)

        @pl.when(step == 0)
        def _():
            acc_ref[...] = jnp.zeros_like(acc_ref)

        t = _dot(h_ref[...], wa_ref[...]) + ba_ref[...]
        t = t / jnp.sqrt(jnp.float32(1.0 + _BN_EPS)) * g_ref[...] + bt_ref[...]
        t = jnp.maximum(t, 0.0)
        t = _dot(t, wb_ref[...]) + bb_ref[...]
        h3 = jnp.maximum(t, 0.0)
        bids = b_ref[0]
        gids = lax.broadcasted_iota(jnp.int32, (_G, _BLK), 0)
        onehot_t = (jnp.broadcast_to(bids, (_G, _BLK)) == gids).astype(
            jnp.float32)
        acc_ref[...] += jnp.dot(onehot_t, h3,
                                preferred_element_type=jnp.float32,
                                precision=lax.Precision.HIGHEST)

        @pl.when(step == _NBLK - 1)
        def _():
            hg = acc_ref[...]
            u = jnp.maximum(_dot(hg, wl1_ref[...]) + bl1_ref[...], 0.0)
            o_ref[...] = _dot(u, wl2_ref[...]) + bl2_ref[...]

    vec = pl.BlockSpec((1, _D), lambda i: (0, 0))
    return pl.pallas_call(
        body,
        grid=(_NBLK,),
        in_specs=[
            pl.BlockSpec((1, 1, _BLK), lambda i: (i, 0, 0)),
            pl.BlockSpec((_BLK, _D), lambda i: (i, 0)),
            pl.BlockSpec((_D, _D), lambda i: (0, 0)),
            vec, vec, vec,
            pl.BlockSpec((_D, _D), lambda i: (0, 0)),
            vec,
            pl.BlockSpec((_D, _D), lambda i: (0, 0)),
            vec,
            pl.BlockSpec((_D, 1), lambda i: (0, 0)),
            pl.BlockSpec((1, 1), lambda i: (0, 0)),
        ],
        out_specs=pl.BlockSpec((_G, 1), lambda i: (0, 0)),
        out_shape=jax.ShapeDtypeStruct((_G, 1), jnp.float32),
        scratch_shapes=[pltpu.VMEM((_G, _D), jnp.float32)],
    )(batch3d, s3, Wa, ba, g, bt, Wb, bb, Wl1, bl1, Wl2, bl2)


def kernel(x, edge_index, batch, W1a, b1a, g1, bt1, W1b, b1b, W2a, b2a, g2,
           bt2, W2b, b2b, W3a, b3a, g3, bt3, W3b, b3b, Wl1, bl1, Wl2, bl2):
    f32 = jnp.float32
    W1a_p = jnp.zeros((16, _D), f32).at[:11].set(W1a)
    r = lambda v: v.reshape(1, _D)

    h0 = jnp.zeros((_NPAD, 16), f32).at[:_N, :11].set(x)
    src = edge_index[0]
    dst = edge_index[1]
    src_p = jnp.zeros((_EROWS_PAD * 128,), jnp.int32).at[:_E].set(src).reshape(
        _EROWS_PAD, 128)
    dst_p = jnp.full((_EROWS_PAD * 128,), -1, jnp.int32).at[:_E].set(
        dst).reshape(_EROWS_PAD, 128)
    edg = jnp.stack([src_p, dst_p], axis=1)
    batch3d = jnp.full((_NPAD,), _G, jnp.int32).at[:_N].set(batch).reshape(
        _NBLK, 1, _BLK)

    s1 = _seg16(h0, edg)
    h1 = _mlp(s1, W1a_p, r(b1a), r(g1), r(bt1), W1b, r(b1b), 16)
    s2 = _seg64(h1, edg)
    h2 = _mlp(s2, W2a, r(b2a), r(g2), r(bt2), W2b, r(b2b), _D)
    s3 = _seg64(h2, edg)
    return _mlp_pool_head(s3, batch3d, W3a, r(b3a), r(g3), r(bt3), W3b,
                          r(b3b), Wl1, r(bl1), Wl2, bl2.reshape(1, 1))
